# Initial kernel scaffold; baseline (speedup 1.0000x reference)
import jax
import jax.numpy as jnp
from jax.experimental import pallas as pl


def kernel(batch_features, batch_labels, features, labels):
    raise NotImplementedError("write your pallas kernel here")


# speedup vs baseline: 4.2659x; 4.2659x over previous
import jax
import jax.numpy as jnp
from jax.experimental import pallas as pl

QUEUE_SIZE = 65536
FEATURE_DIM = 128
BATCH = 16384

BLOCK_ROWS = 2048
GRID = QUEUE_SIZE // BLOCK_ROWS
NB = BATCH // BLOCK_ROWS
LPR = BLOCK_ROWS // FEATURE_DIM


def _copy_kernel(bf_ref, bl_ref, f_ref, l_ref, of_ref, ol_ref):
    i = pl.program_id(0)

    @pl.when(i < NB)
    def _():
        of_ref[...] = bf_ref[...]
        ol_ref[...] = bl_ref[...]

    @pl.when(i >= NB)
    def _():
        of_ref[...] = f_ref[...]
        ol_ref[...] = l_ref[...]


def kernel(batch_features, batch_labels, features, labels):
    bl3 = batch_labels.reshape(NB, LPR, FEATURE_DIM)
    l3 = labels.reshape(GRID, LPR, FEATURE_DIM)

    fspec_new = pl.BlockSpec((BLOCK_ROWS, FEATURE_DIM), lambda i: (i, 0))
    lspec_new = pl.BlockSpec((1, LPR, FEATURE_DIM), lambda i: (i, 0, 0))

    out_f, out_l = pl.pallas_call(
        _copy_kernel,
        grid=(GRID,),
        in_specs=[
            pl.BlockSpec((BLOCK_ROWS, FEATURE_DIM),
                         lambda i: (jnp.minimum(i, NB - 1), 0)),
            pl.BlockSpec((1, LPR, FEATURE_DIM),
                         lambda i: (jnp.minimum(i, NB - 1), 0, 0)),
            pl.BlockSpec((BLOCK_ROWS, FEATURE_DIM),
                         lambda i: (jnp.maximum(i, NB), 0)),
            pl.BlockSpec((1, LPR, FEATURE_DIM),
                         lambda i: (jnp.maximum(i, NB), 0, 0)),
        ],
        out_specs=[fspec_new, lspec_new],
        out_shape=[
            jax.ShapeDtypeStruct((QUEUE_SIZE, FEATURE_DIM), jnp.float32),
            jax.ShapeDtypeStruct((GRID, LPR, FEATURE_DIM), jnp.int32),
        ],
    )(batch_features, bl3, features, l3)

    return out_f, out_l.reshape(QUEUE_SIZE)


# speedup vs baseline: 5.6234x; 1.3182x over previous
import jax
import jax.numpy as jnp
from jax.experimental import pallas as pl

QUEUE_SIZE = 65536
FEATURE_DIM = 128
BATCH = 16384

BLOCK_ROWS = 4096
GRID = QUEUE_SIZE // BLOCK_ROWS
NB = BATCH // BLOCK_ROWS
LPR = BLOCK_ROWS // FEATURE_DIM


def _copy_kernel(bf_ref, bl_ref, f_ref, l_ref, of_ref, ol_ref):
    i = pl.program_id(0)

    @pl.when(i < NB)
    def _():
        of_ref[...] = bf_ref[...]
        ol_ref[...] = bl_ref[...]

    @pl.when(i >= NB)
    def _():
        of_ref[...] = f_ref[...]
        ol_ref[...] = l_ref[...]


def kernel(batch_features, batch_labels, features, labels):
    bl3 = batch_labels.reshape(NB, LPR, FEATURE_DIM)
    l3 = labels.reshape(GRID, LPR, FEATURE_DIM)

    fspec_new = pl.BlockSpec((BLOCK_ROWS, FEATURE_DIM), lambda i: (i, 0))
    lspec_new = pl.BlockSpec((1, LPR, FEATURE_DIM), lambda i: (i, 0, 0))

    out_f, out_l = pl.pallas_call(
        _copy_kernel,
        grid=(GRID,),
        in_specs=[
            pl.BlockSpec((BLOCK_ROWS, FEATURE_DIM),
                         lambda i: (jnp.minimum(i, NB - 1), 0)),
            pl.BlockSpec((1, LPR, FEATURE_DIM),
                         lambda i: (jnp.minimum(i, NB - 1), 0, 0)),
            pl.BlockSpec((BLOCK_ROWS, FEATURE_DIM),
                         lambda i: (jnp.maximum(i, NB), 0)),
            pl.BlockSpec((1, LPR, FEATURE_DIM),
                         lambda i: (jnp.maximum(i, NB), 0, 0)),
        ],
        out_specs=[fspec_new, lspec_new],
        out_shape=[
            jax.ShapeDtypeStruct((QUEUE_SIZE, FEATURE_DIM), jnp.float32),
            jax.ShapeDtypeStruct((GRID, LPR, FEATURE_DIM), jnp.int32),
        ],
    )(batch_features, bl3, features, l3)

    return out_f, out_l.reshape(QUEUE_SIZE)


# speedup vs baseline: 6.2116x; 1.1046x over previous
import jax
import jax.numpy as jnp
from jax.experimental import pallas as pl

QUEUE_SIZE = 65536
FEATURE_DIM = 128
BATCH = 16384

BLOCK_ROWS = 8192
GRID = QUEUE_SIZE // BLOCK_ROWS
NB = BATCH // BLOCK_ROWS
LPR = BLOCK_ROWS // FEATURE_DIM


def _copy_kernel(bf_ref, bl_ref, f_ref, l_ref, of_ref, ol_ref):
    i = pl.program_id(0)

    @pl.when(i < NB)
    def _():
        of_ref[...] = bf_ref[...]
        ol_ref[...] = bl_ref[...]

    @pl.when(i >= NB)
    def _():
        of_ref[...] = f_ref[...]
        ol_ref[...] = l_ref[...]


def kernel(batch_features, batch_labels, features, labels):
    bl3 = batch_labels.reshape(NB, LPR, FEATURE_DIM)
    l3 = labels.reshape(GRID, LPR, FEATURE_DIM)

    fspec_new = pl.BlockSpec((BLOCK_ROWS, FEATURE_DIM), lambda i: (i, 0))
    lspec_new = pl.BlockSpec((1, LPR, FEATURE_DIM), lambda i: (i, 0, 0))

    out_f, out_l = pl.pallas_call(
        _copy_kernel,
        grid=(GRID,),
        in_specs=[
            pl.BlockSpec((BLOCK_ROWS, FEATURE_DIM),
                         lambda i: (jnp.minimum(i, NB - 1), 0)),
            pl.BlockSpec((1, LPR, FEATURE_DIM),
                         lambda i: (jnp.minimum(i, NB - 1), 0, 0)),
            pl.BlockSpec((BLOCK_ROWS, FEATURE_DIM),
                         lambda i: (jnp.maximum(i, NB), 0)),
            pl.BlockSpec((1, LPR, FEATURE_DIM),
                         lambda i: (jnp.maximum(i, NB), 0, 0)),
        ],
        out_specs=[fspec_new, lspec_new],
        out_shape=[
            jax.ShapeDtypeStruct((QUEUE_SIZE, FEATURE_DIM), jnp.float32),
            jax.ShapeDtypeStruct((GRID, LPR, FEATURE_DIM), jnp.int32),
        ],
    )(batch_features, bl3, features, l3)

    return out_f, out_l.reshape(QUEUE_SIZE)
